# baseline (speedup 1.0000x reference)
import functools

import jax
import jax.numpy as jnp
from jax import lax
from jax.experimental import pallas as pl
from jax.experimental.pallas import tpu as pltpu
from jax.experimental.pallas import tpu_sc as plsc

NUM_WORKERS = 32
CHUNK = 128
NBUF = 4
LANES = 16


@functools.lru_cache(maxsize=None)
def _build(b0_total, b1_total, d_model):
    assert b0_total == NUM_WORKERS * CHUNK
    n_tile_cols = b0_total // CHUNK
    q_rows = b0_total // CHUNK * (d_model // 8) * 8
    mesh = plsc.VectorSubcoreMesh(core_axis_name="c", subcore_axis_name="s")

    @functools.partial(
        pl.kernel,
        mesh=mesh,
        out_type=jax.ShapeDtypeStruct((b1_total, q_rows, CHUNK), jnp.float32),
        scratch_types=[
            pltpu.VMEM((b1_total, CHUNK), jnp.int32),
            pltpu.VMEM((d_model, CHUNK), jnp.float32),
        ]
        + [pltpu.VMEM((CHUNK, d_model), jnp.float32) for _ in range(NBUF)]
        + [pltpu.SemaphoreType.DMA for _ in range(NBUF)],
        compiler_params=pltpu.CompilerParams(
            use_tc_tiling_on_sc=False, needs_layout_passes=False
        ),
    )
    def gather_kernel(xt_hbm, table_hbm, out_hbm, idx_v, perm_v, *bufs_sems):
        bufs = bufs_sems[:NBUF]
        sems = bufs_sems[NBUF:]
        wid = lax.axis_index("s") * 2 + lax.axis_index("c")
        pltpu.sync_copy(xt_hbm.at[:, pl.ds(wid * CHUNK, CHUNK)], idx_v)

        def start(b1, slot):
            pltpu.async_copy(
                table_hbm.at[idx_v.at[b1]], bufs[slot], sems[slot]
            )

        def finish(b1, slot):
            pltpu.make_async_copy(
                table_hbm.at[idx_v.at[b1]], bufs[slot], sems[slot]
            ).wait()
            rows = bufs[slot]
            lane = lax.iota(jnp.int32, LANES)

            def perm_body(d, carry):
                for g in range(CHUNK // LANES):
                    vec = plsc.load_gather(
                        rows, [lane + (g * LANES), jnp.full((LANES,), d, jnp.int32)]
                    )
                    perm_v.at[d][pl.ds(g * LANES, LANES)] = vec
                return carry

            lax.fori_loop(0, d_model, perm_body, 0)
            for blk in range(d_model // 8):
                pltpu.sync_copy(
                    perm_v.at[pl.ds(blk * 8, 8)],
                    out_hbm.at[b1, pl.ds((blk * n_tile_cols + wid) * 8, 8)],
                )

        for r in range(NBUF):
            start(r, r)

        def body(g, carry):
            for dr in range(NBUF):
                b1 = g * NBUF + dr
                finish(b1, dr)
                start(b1 + NBUF, dr)
            return carry

        lax.fori_loop(0, b1_total // NBUF - 1, body, 0)

        for b1 in range(b1_total - NBUF, b1_total):
            finish(b1, b1 % NBUF)

    return gather_kernel


def kernel(x, table):
    b0, b1 = x.shape
    d = table.shape[1]
    out = _build(b0, b1, d)(x.T.astype(jnp.int32), table)
    out6 = out.reshape(b1, d // 8, b0 // CHUNK, 8, CHUNK)
    return out6.transpose(2, 4, 0, 1, 3).reshape(b0, b1, d)


# speedup vs baseline: 1.9727x; 1.9727x over previous
import functools

import jax
import jax.numpy as jnp
from jax import lax
from jax.experimental import pallas as pl
from jax.experimental.pallas import tpu as pltpu
from jax.experimental.pallas import tpu_sc as plsc

NUM_WORKERS = 32
CHUNK = 128
NBUF = 4
LANES = 16
PPITCH = 129


@functools.lru_cache(maxsize=None)
def _build(b0_total, b1_total, d_model, row_pad):
    assert b0_total == NUM_WORKERS * CHUNK
    n_tile_cols = b0_total // CHUNK
    q_rows = n_tile_cols * d_model
    mesh = plsc.VectorSubcoreMesh(core_axis_name="c", subcore_axis_name="s")

    @functools.partial(
        pl.kernel,
        mesh=mesh,
        out_type=jax.ShapeDtypeStruct((b1_total, q_rows, CHUNK), jnp.float32),
        scratch_types=[
            pltpu.VMEM((b1_total, CHUNK), jnp.int32),
        ]
        + [pltpu.VMEM((d_model, PPITCH), jnp.float32) for _ in range(2)]
        + [pltpu.SemaphoreType.DMA for _ in range(2)]
        + [pltpu.VMEM((CHUNK, row_pad), jnp.float32) for _ in range(NBUF)]
        + [pltpu.SemaphoreType.DMA for _ in range(NBUF)],
        compiler_params=pltpu.CompilerParams(
            use_tc_tiling_on_sc=False, needs_layout_passes=False
        ),
    )
    def gather_kernel(xt_hbm, table_hbm, out_hbm, idx_v, *scratch):
        perms = scratch[:2]
        wsems = scratch[2:4]
        bufs = scratch[4 : 4 + NBUF]
        sems = scratch[4 + NBUF :]
        wid = lax.axis_index("s") * 2 + lax.axis_index("c")
        pltpu.sync_copy(xt_hbm.at[:, pl.ds(wid * CHUNK, CHUNK)], idx_v)
        lane = lax.iota(jnp.int32, LANES)
        drows = [lane + d0 for d0 in range(0, d_model, LANES)]

        def out_blocks(b1, perm):
            return [
                (
                    perm.at[pl.ds(blk * 8, 8), pl.ds(0, CHUNK)],
                    out_hbm.at[b1, pl.ds((blk * n_tile_cols + wid) * 8, 8)],
                )
                for blk in range(d_model // 8)
            ]

        def start(b1, slot):
            pltpu.async_copy(table_hbm.at[idx_v.at[b1]], bufs[slot], sems[slot])

        def finish(b1, slot, pslot, guard_drain):
            pltpu.make_async_copy(
                table_hbm.at[idx_v.at[b1]], bufs[slot], sems[slot]
            ).wait()
            rows = bufs[slot]
            perm = perms[pslot]

            def drain():
                for src, dst in out_blocks(b1 - 2, perm):
                    pltpu.make_async_copy(src, dst, wsems[pslot]).wait()

            if guard_drain:
                pl.when(b1 >= 2)(drain)
            else:
                drain()

            def perm_body(r8, carry):
                for rr in range(8):
                    b0l = r8 * 8 + rr
                    col = jnp.full((LANES,), b0l, jnp.int32)
                    for di, d0 in enumerate(range(0, d_model, LANES)):
                        vec = rows[b0l, pl.ds(d0, LANES)]
                        plsc.store_scatter(perm, [drows[di], col], vec)
                return carry

            lax.fori_loop(0, CHUNK // 8, perm_body, 0)
            for src, dst in out_blocks(b1, perm):
                pltpu.async_copy(src, dst, wsems[pslot])

        for r in range(NBUF):
            start(r, r)

        def body(g, carry):
            for dr in range(NBUF):
                b1 = g * NBUF + dr
                finish(b1, dr, dr % 2, guard_drain=dr < 2)
                start(b1 + NBUF, dr)
            return carry

        n_main = (b1_total // NBUF) - 1
        lax.fori_loop(0, n_main, body, 0)

        for b1 in range(n_main * NBUF, b1_total):
            finish(b1, b1 % NBUF, b1 % 2, guard_drain=False)

        for b1 in (b1_total - 2, b1_total - 1):
            for src, dst in out_blocks(b1, perms[b1 % 2]):
                pltpu.make_async_copy(src, dst, wsems[b1 % 2]).wait()

    return gather_kernel


def kernel(x, table):
    b0, b1 = x.shape
    d = table.shape[1]
    out = _build(b0, b1, d, d)(x.T.astype(jnp.int32), table)
    out6 = out.reshape(b1, d // 8, b0 // CHUNK, 8, CHUNK)
    return out6.transpose(2, 4, 0, 1, 3).reshape(b0, b1, d)
